# baseline (speedup 1.0000x reference)
import functools

import jax
import jax.numpy as jnp
from jax import lax
from jax.experimental import pallas as pl
from jax.experimental.pallas import tpu as pltpu
from jax.experimental.pallas import tpu_sc as plsc

VOCAB = 1000000
HIDDEN = 64
NUM_CLASSES = 3
BATCH = 4096
HIST = 200

_NC = 2
_NS = 16
_NW = _NC * _NS
_ROWS_PER_W = BATCH // _NW
_NG = HIDDEN // 16
_CHUNK_A = 128
_CHUNK_B = HIST - _CHUNK_A


def _pool_body(idx_hbm, table_hbm, out_hbm, idx_v, rows_v, pooled_v, sem):
    wid = lax.axis_index("s") * _NC + lax.axis_index("c")
    base = wid * _ROWS_PER_W
    pltpu.sync_copy(idx_hbm.at[pl.ds(base, _ROWS_PER_W)], idx_v)

    inv = jnp.float32(1.0 / HIST)

    def row_body(b, carry):
        cp_a = pltpu.async_copy(
            table_hbm.at[idx_v.at[b, pl.ds(0, _CHUNK_A)]],
            rows_v.at[pl.ds(0, _CHUNK_A)], sem)
        cp_b = pltpu.async_copy(
            table_hbm.at[idx_v.at[b, pl.ds(_CHUNK_A, _CHUNK_B)]],
            rows_v.at[pl.ds(_CHUNK_A, _CHUNK_B)], sem)
        cp_a.wait()
        cp_b.wait()

        def acc_body(j, accs):
            return tuple(accs[g] + rows_v[j, pl.ds(g * 16, 16)]
                         for g in range(_NG))

        accs = lax.fori_loop(
            0, HIST, acc_body,
            tuple(jnp.zeros((16,), jnp.float32) for _ in range(_NG)))
        for g in range(_NG):
            pooled_v[b, pl.ds(g * 16, 16)] = accs[g] * inv
        return carry

    lax.fori_loop(0, _ROWS_PER_W, row_body, jnp.int32(0))
    pltpu.sync_copy(pooled_v, out_hbm.at[pl.ds(base, _ROWS_PER_W)])


_pool = functools.partial(
    pl.kernel,
    out_type=jax.ShapeDtypeStruct((BATCH, HIDDEN), jnp.float32),
    mesh=plsc.VectorSubcoreMesh(core_axis_name="c", subcore_axis_name="s"),
    scratch_types=[
        pltpu.VMEM((_ROWS_PER_W, HIST), jnp.int32),
        pltpu.VMEM((HIST, HIDDEN), jnp.float32),
        pltpu.VMEM((_ROWS_PER_W, HIDDEN), jnp.float32),
        pltpu.SemaphoreType.DMA,
    ],
    compiler_params=pltpu.CompilerParams(use_tc_tiling_on_sc=False),
)(_pool_body)


def _mm_body(x_ref, w_ref, b_ref, o_ref):
    o_ref[...] = (
        jnp.dot(x_ref[...], w_ref[...], preferred_element_type=jnp.float32)
        + b_ref[...])


_matmul = pl.pallas_call(
    _mm_body,
    out_shape=jax.ShapeDtypeStruct((BATCH, NUM_CLASSES), jnp.float32),
)


def kernel(input_ids, embedding, W, b):
    idx = input_ids.astype(jnp.int32)
    pooled = _pool(idx, embedding)
    return _matmul(pooled, W, b.reshape(1, NUM_CLASSES))


# speedup vs baseline: 1.2356x; 1.2356x over previous
import functools

import jax
import jax.numpy as jnp
from jax import lax
from jax.experimental import pallas as pl
from jax.experimental.pallas import tpu as pltpu
from jax.experimental.pallas import tpu_sc as plsc

VOCAB = 1000000
HIDDEN = 64
NUM_CLASSES = 3
BATCH = 4096
HIST = 200

_NC = 2
_NS = 16
_NW = _NC * _NS
_ROWS_PER_W = BATCH // _NW
_NG = HIDDEN // 16
_NBUF = 4
_UNROLL = 8
_CHUNK_A = 128
_CHUNK_B = HIST - _CHUNK_A


def _pool_body(idx_hbm, table_hbm, out_hbm, idx_v, rows_v, pooled_v, *sems):
    wid = lax.axis_index("s") * _NC + lax.axis_index("c")
    base = wid * _ROWS_PER_W
    pltpu.sync_copy(idx_hbm.at[pl.ds(base, _ROWS_PER_W)], idx_v)

    inv = jnp.float32(1.0 / HIST)

    def issue(b, k):
        pltpu.async_copy(
            table_hbm.at[idx_v.at[b, pl.ds(0, _CHUNK_A)]],
            rows_v.at[k, pl.ds(0, _CHUNK_A)], sems[k])
        pltpu.async_copy(
            table_hbm.at[idx_v.at[b, pl.ds(_CHUNK_A, _CHUNK_B)]],
            rows_v.at[k, pl.ds(_CHUNK_A, _CHUNK_B)], sems[k])

    def wait_row(k):
        pltpu.make_async_copy(
            table_hbm.at[pl.ds(0, HIST)], rows_v.at[k], sems[k]).wait()

    for k in range(_NBUF):
        issue(k, k)

    def quad_body(p, carry):
        for k in range(_NBUF):
            b = p * _NBUF + k
            wait_row(k)

            def acc_body(j, accs, _k=k):
                for u in range(_UNROLL):
                    accs = tuple(
                        accs[g] + rows_v[_k, j * _UNROLL + u, pl.ds(g * 16, 16)]
                        for g in range(_NG))
                return accs

            accs = lax.fori_loop(
                0, HIST // _UNROLL, acc_body,
                tuple(jnp.zeros((16,), jnp.float32) for _ in range(_NG)))

            @pl.when(b + _NBUF < _ROWS_PER_W)
            def _():
                issue(b + _NBUF, k)

            for g in range(_NG):
                pooled_v[b, pl.ds(g * 16, 16)] = accs[g] * inv
        return carry

    lax.fori_loop(0, _ROWS_PER_W // _NBUF, quad_body, jnp.int32(0))
    pltpu.sync_copy(pooled_v, out_hbm.at[pl.ds(base, _ROWS_PER_W)])


_pool = functools.partial(
    pl.kernel,
    out_type=jax.ShapeDtypeStruct((BATCH, HIDDEN), jnp.float32),
    mesh=plsc.VectorSubcoreMesh(core_axis_name="c", subcore_axis_name="s"),
    scratch_types=[
        pltpu.VMEM((_ROWS_PER_W, HIST), jnp.int32),
        pltpu.VMEM((_NBUF, HIST, HIDDEN), jnp.float32),
        pltpu.VMEM((_ROWS_PER_W, HIDDEN), jnp.float32),
    ] + [pltpu.SemaphoreType.DMA] * _NBUF,
    compiler_params=pltpu.CompilerParams(use_tc_tiling_on_sc=False),
)(_pool_body)


def _mm_body(x_ref, w_ref, b_ref, o_ref):
    o_ref[...] = (
        jnp.dot(x_ref[...], w_ref[...], preferred_element_type=jnp.float32)
        + b_ref[...])


_matmul = pl.pallas_call(
    _mm_body,
    out_shape=jax.ShapeDtypeStruct((BATCH, NUM_CLASSES), jnp.float32),
)


def kernel(input_ids, embedding, W, b):
    idx = input_ids.astype(jnp.int32)
    pooled = _pool(idx, embedding)
    return _matmul(pooled, W, b.reshape(1, NUM_CLASSES))


# speedup vs baseline: 1.3632x; 1.1033x over previous
import functools

import jax
import jax.numpy as jnp
from jax import lax
from jax.experimental import pallas as pl
from jax.experimental.pallas import tpu as pltpu
from jax.experimental.pallas import tpu_sc as plsc

VOCAB = 1000000
HIDDEN = 64
NUM_CLASSES = 3
BATCH = 4096
HIST = 200

_PADC = 16
_PACK = 128 // _PADC
_VCHUNK = 16384
_GRID = -(-VOCAB // _VCHUNK)

_NC = 2
_NS = 16
_NW = _NC * _NS
_ROWS_PER_W = BATCH // _NW
_NBUF = 4
_UNROLL = 8
_CHUNK_A = 128
_CHUNK_B = HIST - _CHUNK_A


def _tw_body(et_ref, w_ref, b_ref, o_ref):
    tw = lax.dot_general(
        et_ref[...], w_ref[...],
        dimension_numbers=(((0,), (0,)), ((), ())),
        preferred_element_type=jnp.float32)
    o_ref[...] = tw + b_ref[...]


_tw_matmul = pl.pallas_call(
    _tw_body,
    grid=(_GRID,),
    in_specs=[
        pl.BlockSpec((HIDDEN, _VCHUNK), lambda i: (0, i)),
        pl.BlockSpec((HIDDEN, _PADC), lambda i: (0, 0)),
        pl.BlockSpec((1, _PADC), lambda i: (0, 0)),
    ],
    out_specs=pl.BlockSpec((_VCHUNK, _PADC), lambda i: (i, 0)),
    out_shape=jax.ShapeDtypeStruct((_GRID * _VCHUNK, _PADC), jnp.float32),
)


def _pool_body(idx_hbm, tw_hbm, out_hbm, idx_v, rows_v, pooled_v, *sems):
    wid = lax.axis_index("s") * _NC + lax.axis_index("c")
    base = wid * _ROWS_PER_W
    pltpu.sync_copy(idx_hbm.at[pl.ds(base, _ROWS_PER_W)], idx_v)

    def issue(b, k):
        pltpu.async_copy(
            tw_hbm.at[idx_v.at[b, pl.ds(0, _CHUNK_A)]],
            rows_v.at[k, pl.ds(0, _CHUNK_A)], sems[k])
        pltpu.async_copy(
            tw_hbm.at[idx_v.at[b, pl.ds(_CHUNK_A, _CHUNK_B)]],
            rows_v.at[k, pl.ds(_CHUNK_A, _CHUNK_B)], sems[k])

    def wait_row(k):
        pltpu.make_async_copy(
            tw_hbm.at[pl.ds(0, HIST)], rows_v.at[k], sems[k]).wait()

    for k in range(_NBUF):
        issue(k, k)

    def quad_body(p, carry):
        for k in range(_NBUF):
            b = p * _NBUF + k
            wait_row(k)

            def acc_body(j, acc, _k=k):
                for u in range(_UNROLL):
                    acc = acc + rows_v[_k, j * _UNROLL + u, :]
                return acc

            acc = lax.fori_loop(
                0, HIST // _UNROLL, acc_body, jnp.zeros((_PADC,), jnp.float32))

            @pl.when(b + _NBUF < _ROWS_PER_W)
            def _():
                issue(b + _NBUF, k)

            pooled_v[b, :] = acc
        return carry

    lax.fori_loop(0, _ROWS_PER_W // _NBUF, quad_body, jnp.int32(0))
    pltpu.sync_copy(pooled_v, out_hbm.at[pl.ds(base, _ROWS_PER_W)])


_pool = functools.partial(
    pl.kernel,
    out_type=jax.ShapeDtypeStruct((BATCH, _PADC), jnp.float32),
    mesh=plsc.VectorSubcoreMesh(core_axis_name="c", subcore_axis_name="s"),
    scratch_types=[
        pltpu.VMEM((_ROWS_PER_W, HIST), jnp.int32),
        pltpu.VMEM((_NBUF, HIST, _PADC), jnp.float32),
        pltpu.VMEM((_ROWS_PER_W, _PADC), jnp.float32),
    ] + [pltpu.SemaphoreType.DMA] * _NBUF,
    compiler_params=pltpu.CompilerParams(use_tc_tiling_on_sc=False),
)(_pool_body)


def kernel(input_ids, embedding, W, b):
    idx = input_ids.astype(jnp.int32)
    et = embedding.T
    w_scaled = jnp.pad(W, ((0, 0), (0, _PADC - NUM_CLASSES))) * (1.0 / HIST)
    b_scaled = (jnp.pad(b, (0, _PADC - NUM_CLASSES)) * (1.0 / HIST)
                ).reshape(1, _PADC)
    tw = _tw_matmul(et, w_scaled, b_scaled)
    pooled = _pool(idx, tw)
    return pooled[:, :NUM_CLASSES]


# speedup vs baseline: 2.5232x; 1.8509x over previous
import functools

import jax
import jax.numpy as jnp
from jax import lax
from jax.experimental import pallas as pl
from jax.experimental.pallas import tpu as pltpu
from jax.experimental.pallas import tpu_sc as plsc

VOCAB = 1000000
HIDDEN = 64
NUM_CLASSES = 3
BATCH = 4096
HIST = 200

_PADC = 16
_PACK = 128 // _PADC
_VCHUNK = 16384
_STRIPE = _VCHUNK // _PACK
_GRID = -(-VOCAB // _VCHUNK)

_NC = 2
_NS = 16
_NW = _NC * _NS
_ROWS_PER_W = BATCH // _NW
_IDX_PER_W = _ROWS_PER_W * HIST
_NBUF = 4
_UNROLL = 8
_CHUNK_A = 128
_CHUNK_B = HIST - _CHUNK_A


def _tw_body(et_ref, w_ref, b_ref, o_ref):
    tw = lax.dot_general(
        et_ref[...], w_ref[...],
        dimension_numbers=(((0,), (0,)), ((), ())),
        preferred_element_type=jnp.float32)
    tw = tw + b_ref[...]
    for s in range(_PACK):
        o_ref[:, _PADC * s:_PADC * (s + 1)] = (
            tw[_STRIPE * s:_STRIPE * (s + 1), :])


_tw_matmul = pl.pallas_call(
    _tw_body,
    grid=(_GRID,),
    in_specs=[
        pl.BlockSpec((HIDDEN, _VCHUNK), lambda i: (0, i)),
        pl.BlockSpec((HIDDEN, _PADC), lambda i: (0, 0)),
        pl.BlockSpec((1, _PADC), lambda i: (0, 0)),
    ],
    out_specs=pl.BlockSpec((_STRIPE, 128), lambda i: (i, 0)),
    out_shape=jax.ShapeDtypeStruct((_GRID * _STRIPE, 128), jnp.float32),
)


def _pool_body(idx_hbm, tw_hbm, out_hbm, idx_v, rows_v, pooled_v, *sems):
    wid = lax.axis_index("s") * _NC + lax.axis_index("c")
    base = wid * _ROWS_PER_W
    pltpu.sync_copy(idx_hbm.at[pl.ds(base * HIST, _IDX_PER_W)], idx_v)

    def xf_body(i, carry):
        v = idx_v[pl.ds(i * 16, 16)]
        packed = ((v >> 14) << 14) | ((v & (_STRIPE - 1)) << 3) \
            | ((v >> 11) & (_PACK - 1))
        idx_v[pl.ds(i * 16, 16)] = packed
        return carry

    lax.fori_loop(0, _IDX_PER_W // 16, xf_body, jnp.int32(0))

    def issue(b, k):
        pltpu.async_copy(
            tw_hbm.at[idx_v.at[pl.ds(b * HIST, _CHUNK_A)]],
            rows_v.at[k, pl.ds(0, _CHUNK_A)], sems[k])
        pltpu.async_copy(
            tw_hbm.at[idx_v.at[pl.ds(b * HIST + _CHUNK_A, _CHUNK_B)]],
            rows_v.at[k, pl.ds(_CHUNK_A, _CHUNK_B)], sems[k])

    def wait_row(k):
        pltpu.make_async_copy(
            tw_hbm.at[pl.ds(0, HIST)], rows_v.at[k], sems[k]).wait()

    for k in range(_NBUF):
        issue(k, k)

    def quad_body(p, carry):
        for k in range(_NBUF):
            b = p * _NBUF + k
            wait_row(k)

            def acc_body(j, acc, _k=k):
                for u in range(_UNROLL):
                    acc = acc + rows_v[_k, j * _UNROLL + u, :]
                return acc

            acc = lax.fori_loop(
                0, HIST // _UNROLL, acc_body, jnp.zeros((_PADC,), jnp.float32))

            @pl.when(b + _NBUF < _ROWS_PER_W)
            def _():
                issue(b + _NBUF, k)

            pooled_v[b, :] = acc
        return carry

    lax.fori_loop(0, _ROWS_PER_W // _NBUF, quad_body, jnp.int32(0))
    pltpu.sync_copy(pooled_v, out_hbm.at[pl.ds(base, _ROWS_PER_W)])


_pool = functools.partial(
    pl.kernel,
    out_type=jax.ShapeDtypeStruct((BATCH, _PADC), jnp.float32),
    mesh=plsc.VectorSubcoreMesh(core_axis_name="c", subcore_axis_name="s"),
    scratch_types=[
        pltpu.VMEM((_IDX_PER_W,), jnp.int32),
        pltpu.VMEM((_NBUF, HIST, _PADC), jnp.float32),
        pltpu.VMEM((_ROWS_PER_W, _PADC), jnp.float32),
    ] + [pltpu.SemaphoreType.DMA] * _NBUF,
    compiler_params=pltpu.CompilerParams(use_tc_tiling_on_sc=False),
)(_pool_body)


def kernel(input_ids, embedding, W, b):
    idx = input_ids.astype(jnp.int32).reshape(-1)
    et = embedding.T
    w_scaled = jnp.pad(W, ((0, 0), (0, _PADC - NUM_CLASSES))) * (1.0 / HIST)
    b_scaled = (jnp.pad(b, (0, _PADC - NUM_CLASSES)) * (1.0 / HIST)
                ).reshape(1, _PADC)
    tw_packed = _tw_matmul(et, w_scaled, b_scaled)
    tw = tw_packed.reshape(-1, _PADC)
    pooled = _pool(idx, tw)
    return pooled[:, :NUM_CLASSES]


# speedup vs baseline: 2.5247x; 1.0006x over previous
import functools

import jax
import jax.numpy as jnp
from jax import lax
from jax.experimental import pallas as pl
from jax.experimental.pallas import tpu as pltpu
from jax.experimental.pallas import tpu_sc as plsc

VOCAB = 1000000
HIDDEN = 64
NUM_CLASSES = 3
BATCH = 4096
HIST = 200

_PADC = 16
_PACK = 128 // _PADC
_VCHUNK = 32768
_STRIPE = _VCHUNK // _PACK
_VSHIFT = _VCHUNK.bit_length() - 1
_SSHIFT = _STRIPE.bit_length() - 1
_GRID = -(-VOCAB // _VCHUNK)

_NC = 2
_NS = 16
_NW = _NC * _NS
_ROWS_PER_W = BATCH // _NW
_IDX_PER_W = _ROWS_PER_W * HIST
_NBUF = 4
_UNROLL = 8
_CHUNK_A = 128
_CHUNK_B = HIST - _CHUNK_A


def _tw_body(et_ref, w_ref, b_ref, o_ref):
    tw = lax.dot_general(
        et_ref[...], w_ref[...],
        dimension_numbers=(((0,), (0,)), ((), ())),
        preferred_element_type=jnp.float32)
    tw = tw + b_ref[...]
    for s in range(_PACK):
        o_ref[:, _PADC * s:_PADC * (s + 1)] = (
            tw[_STRIPE * s:_STRIPE * (s + 1), :])


_tw_matmul = pl.pallas_call(
    _tw_body,
    grid=(_GRID,),
    in_specs=[
        pl.BlockSpec((HIDDEN, _VCHUNK), lambda i: (0, i)),
        pl.BlockSpec((HIDDEN, _PADC), lambda i: (0, 0)),
        pl.BlockSpec((1, _PADC), lambda i: (0, 0)),
    ],
    out_specs=pl.BlockSpec((_STRIPE, 128), lambda i: (i, 0)),
    out_shape=jax.ShapeDtypeStruct((_GRID * _STRIPE, 128), jnp.float32),
)


def _pool_body(idx_hbm, tw_hbm, out_hbm, idx_v, rows_v, pooled_v, *sems):
    wid = lax.axis_index("s") * _NC + lax.axis_index("c")
    base = wid * _ROWS_PER_W
    pltpu.sync_copy(idx_hbm.at[pl.ds(base * HIST, _IDX_PER_W)], idx_v)

    def xf_body(i, carry):
        v = idx_v[pl.ds(i * 16, 16)]
        packed = ((v >> _VSHIFT) << _VSHIFT) | ((v & (_STRIPE - 1)) << 3) \
            | ((v >> _SSHIFT) & (_PACK - 1))
        idx_v[pl.ds(i * 16, 16)] = packed
        return carry

    lax.fori_loop(0, _IDX_PER_W // 16, xf_body, jnp.int32(0))

    def issue(b, k):
        pltpu.async_copy(
            tw_hbm.at[idx_v.at[pl.ds(b * HIST, _CHUNK_A)]],
            rows_v.at[k, pl.ds(0, _CHUNK_A)], sems[k])
        pltpu.async_copy(
            tw_hbm.at[idx_v.at[pl.ds(b * HIST + _CHUNK_A, _CHUNK_B)]],
            rows_v.at[k, pl.ds(_CHUNK_A, _CHUNK_B)], sems[k])

    def wait_row(k):
        pltpu.make_async_copy(
            tw_hbm.at[pl.ds(0, HIST)], rows_v.at[k], sems[k]).wait()

    for k in range(_NBUF):
        issue(k, k)

    def quad_body(p, carry):
        for k in range(_NBUF):
            b = p * _NBUF + k
            wait_row(k)

            def acc_body(j, acc, _k=k):
                for u in range(_UNROLL):
                    acc = acc + rows_v[_k, j * _UNROLL + u, :]
                return acc

            acc = lax.fori_loop(
                0, HIST // _UNROLL, acc_body, jnp.zeros((_PADC,), jnp.float32))

            @pl.when(b + _NBUF < _ROWS_PER_W)
            def _():
                issue(b + _NBUF, k)

            pooled_v[b, :] = acc
        return carry

    lax.fori_loop(0, _ROWS_PER_W // _NBUF, quad_body, jnp.int32(0))
    pltpu.sync_copy(pooled_v, out_hbm.at[pl.ds(base, _ROWS_PER_W)])


_pool = functools.partial(
    pl.kernel,
    out_type=jax.ShapeDtypeStruct((BATCH, _PADC), jnp.float32),
    mesh=plsc.VectorSubcoreMesh(core_axis_name="c", subcore_axis_name="s"),
    scratch_types=[
        pltpu.VMEM((_IDX_PER_W,), jnp.int32),
        pltpu.VMEM((_NBUF, HIST, _PADC), jnp.float32),
        pltpu.VMEM((_ROWS_PER_W, _PADC), jnp.float32),
    ] + [pltpu.SemaphoreType.DMA] * _NBUF,
    compiler_params=pltpu.CompilerParams(use_tc_tiling_on_sc=False),
)(_pool_body)


def kernel(input_ids, embedding, W, b):
    idx = input_ids.astype(jnp.int32).reshape(-1)
    et = embedding.T
    w_scaled = jnp.pad(W, ((0, 0), (0, _PADC - NUM_CLASSES))) * (1.0 / HIST)
    b_scaled = (jnp.pad(b, (0, _PADC - NUM_CLASSES)) * (1.0 / HIST)
                ).reshape(1, _PADC)
    tw_packed = _tw_matmul(et, w_scaled, b_scaled)
    tw = tw_packed.reshape(-1, _PADC)
    pooled = _pool(idx, tw)
    return pooled[:, :NUM_CLASSES]


# speedup vs baseline: 2.5642x; 1.0157x over previous
import functools

import jax
import jax.numpy as jnp
from jax import lax
from jax.experimental import pallas as pl
from jax.experimental.pallas import tpu as pltpu
from jax.experimental.pallas import tpu_sc as plsc

VOCAB = 1000000
HIDDEN = 64
NUM_CLASSES = 3
BATCH = 4096
HIST = 200

_PADC = 16
_PACK = 128 // _PADC
_VCHUNK = 16384
_STRIPE = _VCHUNK // _PACK
_VSHIFT = _VCHUNK.bit_length() - 1
_SSHIFT = _STRIPE.bit_length() - 1
_GRID = -(-VOCAB // _VCHUNK)

_NC = 2
_NS = 16
_NW = _NC * _NS
_ROWS_PER_W = BATCH // _NW
_IDX_PER_W = _ROWS_PER_W * HIST
_NBUF = 8
_UNROLL = 8
_CHUNK_A = 128
_CHUNK_B = HIST - _CHUNK_A


def _tw_body(et_ref, w_ref, b_ref, o_ref):
    for s in range(_PACK):
        blk = lax.dot_general(
            et_ref[:, _STRIPE * s:_STRIPE * (s + 1)], w_ref[...],
            dimension_numbers=(((0,), (0,)), ((), ())),
            preferred_element_type=jnp.float32)
        o_ref[:, _PADC * s:_PADC * (s + 1)] = blk + b_ref[...]


_tw_matmul = pl.pallas_call(
    _tw_body,
    grid=(_GRID,),
    in_specs=[
        pl.BlockSpec((HIDDEN, _VCHUNK), lambda i: (0, i)),
        pl.BlockSpec((HIDDEN, _PADC), lambda i: (0, 0)),
        pl.BlockSpec((1, _PADC), lambda i: (0, 0)),
    ],
    out_specs=pl.BlockSpec((_STRIPE, 128), lambda i: (i, 0)),
    out_shape=jax.ShapeDtypeStruct((_GRID * _STRIPE, 128), jnp.float32),
)


def _pool_body(idx_hbm, tw_hbm, out_hbm, idx_v, rows_v, pooled_v, *sems):
    wid = lax.axis_index("s") * _NC + lax.axis_index("c")
    base = wid * _ROWS_PER_W
    pltpu.sync_copy(idx_hbm.at[pl.ds(base * HIST, _IDX_PER_W)], idx_v)

    def xf_body(i, carry):
        v = idx_v[pl.ds(i * 16, 16)]
        packed = ((v >> _VSHIFT) << _VSHIFT) | ((v & (_STRIPE - 1)) << 3) \
            | ((v >> _SSHIFT) & (_PACK - 1))
        idx_v[pl.ds(i * 16, 16)] = packed
        return carry

    lax.fori_loop(0, _IDX_PER_W // 16, xf_body, jnp.int32(0))

    def issue(b, k):
        pltpu.async_copy(
            tw_hbm.at[idx_v.at[pl.ds(b * HIST, _CHUNK_A)]],
            rows_v.at[k, pl.ds(0, _CHUNK_A)], sems[k])
        pltpu.async_copy(
            tw_hbm.at[idx_v.at[pl.ds(b * HIST + _CHUNK_A, _CHUNK_B)]],
            rows_v.at[k, pl.ds(_CHUNK_A, _CHUNK_B)], sems[k])

    def wait_row(k):
        pltpu.make_async_copy(
            tw_hbm.at[pl.ds(0, HIST)], rows_v.at[k], sems[k]).wait()

    for k in range(_NBUF):
        issue(k, k)

    def quad_body(p, carry):
        for k in range(_NBUF):
            b = p * _NBUF + k
            wait_row(k)

            def acc_body(j, acc, _k=k):
                for u in range(_UNROLL):
                    acc = acc + rows_v[_k, j * _UNROLL + u, :]
                return acc

            acc = lax.fori_loop(
                0, HIST // _UNROLL, acc_body, jnp.zeros((_PADC,), jnp.float32))

            @pl.when(b + _NBUF < _ROWS_PER_W)
            def _():
                issue(b + _NBUF, k)

            pooled_v[b, :] = acc
        return carry

    lax.fori_loop(0, _ROWS_PER_W // _NBUF, quad_body, jnp.int32(0))
    pltpu.sync_copy(pooled_v, out_hbm.at[pl.ds(base, _ROWS_PER_W)])


_pool = functools.partial(
    pl.kernel,
    out_type=jax.ShapeDtypeStruct((BATCH, _PADC), jnp.float32),
    mesh=plsc.VectorSubcoreMesh(core_axis_name="c", subcore_axis_name="s"),
    scratch_types=[
        pltpu.VMEM((_IDX_PER_W,), jnp.int32),
        pltpu.VMEM((_NBUF, HIST, _PADC), jnp.float32),
        pltpu.VMEM((_ROWS_PER_W, _PADC), jnp.float32),
    ] + [pltpu.SemaphoreType.DMA] * _NBUF,
    compiler_params=pltpu.CompilerParams(use_tc_tiling_on_sc=False),
)(_pool_body)


def kernel(input_ids, embedding, W, b):
    idx = input_ids.astype(jnp.int32).reshape(-1)
    et = embedding.T
    w_scaled = jnp.pad(W, ((0, 0), (0, _PADC - NUM_CLASSES))) * (1.0 / HIST)
    b_scaled = (jnp.pad(b, (0, _PADC - NUM_CLASSES)) * (1.0 / HIST)
                ).reshape(1, _PADC)
    tw_packed = _tw_matmul(et, w_scaled, b_scaled)
    tw = tw_packed.reshape(-1, _PADC)
    pooled = _pool(idx, tw)
    return pooled[:, :NUM_CLASSES]


# speedup vs baseline: 2.6177x; 1.0208x over previous
import functools

import jax
import jax.numpy as jnp
from jax import lax
from jax.experimental import pallas as pl
from jax.experimental.pallas import tpu as pltpu
from jax.experimental.pallas import tpu_sc as plsc

VOCAB = 1000000
HIDDEN = 64
NUM_CLASSES = 3
BATCH = 4096
HIST = 200

_PADC = 16
_PACK = 128 // _PADC
_VCHUNK = 16384
_STRIPE = _VCHUNK // _PACK
_VSHIFT = _VCHUNK.bit_length() - 1
_SSHIFT = _STRIPE.bit_length() - 1
_GRID = -(-VOCAB // _VCHUNK)

_NC = 2
_NS = 16
_NW = _NC * _NS
_ROWS_PER_W = BATCH // _NW
_IDX_PER_W = _ROWS_PER_W * HIST
_NBUF = 8
_UNROLL = 10
_CHUNK_A = 128
_CHUNK_B = HIST - _CHUNK_A


def _tw_body(et_ref, w_ref, b_ref, o_ref):
    for s in range(_PACK):
        blk = lax.dot_general(
            et_ref[:, _STRIPE * s:_STRIPE * (s + 1)], w_ref[...],
            dimension_numbers=(((0,), (0,)), ((), ())),
            preferred_element_type=jnp.float32)
        o_ref[:, _PADC * s:_PADC * (s + 1)] = blk + b_ref[...]


_tw_matmul = pl.pallas_call(
    _tw_body,
    grid=(_GRID,),
    in_specs=[
        pl.BlockSpec((HIDDEN, _VCHUNK), lambda i: (0, i)),
        pl.BlockSpec((HIDDEN, _PADC), lambda i: (0, 0)),
        pl.BlockSpec((1, _PADC), lambda i: (0, 0)),
    ],
    out_specs=pl.BlockSpec((_STRIPE, 128), lambda i: (i, 0)),
    out_shape=jax.ShapeDtypeStruct((_GRID * _STRIPE, 128), jnp.float32),
)


def _pool_body(idx_hbm, tw_hbm, out_hbm, idx_v, rows_v, pooled_v, *sems):
    wid = lax.axis_index("s") * _NC + lax.axis_index("c")
    base = wid * _ROWS_PER_W
    pltpu.sync_copy(idx_hbm.at[pl.ds(base * HIST, _IDX_PER_W)], idx_v)

    def xf_body(i, carry):
        for u in range(4):
            off = (i * 4 + u) * 16
            v = idx_v[pl.ds(off, 16)]
            packed = ((v >> _VSHIFT) << _VSHIFT) \
                | ((v & (_STRIPE - 1)) << 3) \
                | ((v >> _SSHIFT) & (_PACK - 1))
            idx_v[pl.ds(off, 16)] = packed
        return carry

    lax.fori_loop(0, _IDX_PER_W // 64, xf_body, jnp.int32(0))

    def issue(b, k):
        pltpu.async_copy(
            tw_hbm.at[idx_v.at[pl.ds(b * HIST, _CHUNK_A)]],
            rows_v.at[k, pl.ds(0, _CHUNK_A)], sems[k])
        pltpu.async_copy(
            tw_hbm.at[idx_v.at[pl.ds(b * HIST + _CHUNK_A, _CHUNK_B)]],
            rows_v.at[k, pl.ds(_CHUNK_A, _CHUNK_B)], sems[k])

    def wait_row(k):
        pltpu.make_async_copy(
            tw_hbm.at[pl.ds(0, HIST)], rows_v.at[k], sems[k]).wait()

    for k in range(_NBUF):
        issue(k, k)

    def quad_body(p, carry):
        for k in range(_NBUF):
            b = p * _NBUF + k
            wait_row(k)

            def acc_body(j, acc, _k=k):
                for u in range(_UNROLL):
                    acc = acc + rows_v[_k, j * _UNROLL + u, :]
                return acc

            acc = lax.fori_loop(
                0, HIST // _UNROLL, acc_body, jnp.zeros((_PADC,), jnp.float32))

            @pl.when(b + _NBUF < _ROWS_PER_W)
            def _():
                issue(b + _NBUF, k)

            pooled_v[b, :] = acc
        return carry

    lax.fori_loop(0, _ROWS_PER_W // _NBUF, quad_body, jnp.int32(0))
    pltpu.sync_copy(pooled_v, out_hbm.at[pl.ds(base, _ROWS_PER_W)])


_pool = functools.partial(
    pl.kernel,
    out_type=jax.ShapeDtypeStruct((BATCH, _PADC), jnp.float32),
    mesh=plsc.VectorSubcoreMesh(core_axis_name="c", subcore_axis_name="s"),
    scratch_types=[
        pltpu.VMEM((_IDX_PER_W,), jnp.int32),
        pltpu.VMEM((_NBUF, HIST, _PADC), jnp.float32),
        pltpu.VMEM((_ROWS_PER_W, _PADC), jnp.float32),
    ] + [pltpu.SemaphoreType.DMA] * _NBUF,
    compiler_params=pltpu.CompilerParams(use_tc_tiling_on_sc=False),
)(_pool_body)


def kernel(input_ids, embedding, W, b):
    idx = input_ids.astype(jnp.int32).reshape(-1)
    et = embedding.T
    w_scaled = jnp.pad(W, ((0, 0), (0, _PADC - NUM_CLASSES))) * (1.0 / HIST)
    b_scaled = (jnp.pad(b, (0, _PADC - NUM_CLASSES)) * (1.0 / HIST)
                ).reshape(1, _PADC)
    tw_packed = _tw_matmul(et, w_scaled, b_scaled)
    tw = tw_packed.reshape(-1, _PADC)
    pooled = _pool(idx, tw)
    return pooled[:, :NUM_CLASSES]
